# Initial kernel scaffold; baseline (speedup 1.0000x reference)
import jax
import jax.numpy as jnp
from jax.experimental import pallas as pl


def kernel(time, value, var_id, category_mask, W1_t, b1_t, W2_t, W1_v, b1_v, W2_v, emb_table):
    raise NotImplementedError("write your pallas kernel here")


# speedup vs baseline: 1.2627x; 1.2627x over previous
import functools

import jax
import jax.numpy as jnp
from jax import lax
from jax.experimental import pallas as pl
from jax.experimental.pallas import tpu as pltpu
from jax.experimental.pallas import tpu_sc as plsc

B, L = 4096, 200
NTOK = B * L
EMB_DIM = 64
HID = 8

NC, NS = 2, 16
NW = NC * NS
TOK_PER_W = NTOK // NW
CHUNK = 1024
STREAM = 128
N_STEPS = TOK_PER_W // CHUNK
N_SUB = CHUNK // STREAM


def _sc_gather_body(table_hbm, idx_hbm, out_hbm, idx_v, rows_v, sem):
    wid = lax.axis_index("s") * NC + lax.axis_index("c")
    base = wid * TOK_PER_W

    def step(i, _):
        off = base + i * CHUNK
        pltpu.sync_copy(idx_hbm.at[pl.ds(off, CHUNK)], idx_v)
        copies = []
        for j in range(N_SUB):
            copies.append(
                pltpu.async_copy(
                    table_hbm.at[idx_v.at[pl.ds(j * STREAM, STREAM)]],
                    rows_v.at[pl.ds(j * STREAM, STREAM)],
                    sem,
                )
            )
        for c in copies:
            c.wait()
        pltpu.sync_copy(rows_v, out_hbm.at[pl.ds(off, CHUNK)])
        return ()

    lax.fori_loop(0, N_STEPS, step, (), unroll=False)


def _sc_gather(table, idx_flat):
    mesh = plsc.VectorSubcoreMesh(core_axis_name="c", subcore_axis_name="s")
    k = functools.partial(
        pl.kernel,
        mesh=mesh,
        out_type=jax.ShapeDtypeStruct((NTOK, EMB_DIM), jnp.float32),
        scratch_types=[
            pltpu.VMEM((CHUNK,), jnp.int32),
            pltpu.VMEM((CHUNK, EMB_DIM), jnp.float32),
            pltpu.SemaphoreType.DMA,
        ],
        compiler_params=pltpu.CompilerParams(use_tc_tiling_on_sc=False),
    )(_sc_gather_body)
    return k(table, idx_flat)


BLK = 2048
GRID = NTOK // BLK


def _tc_body(xt, xv, cmf, vid, w1t, b1t, w1v, b1v, wcat, gath, out, mask):
    t = xt[...]
    v = xv[...]
    h_t = jnp.tanh(t * w1t[...] + b1t[...])
    h_v = jnp.tanh(v * w1v[...] + b1v[...]) * cmf[...]
    h = jnp.concatenate([h_t, h_v], axis=1)
    cve = jnp.dot(h, wcat[...], preferred_element_type=jnp.float32)
    out[...] = cve + gath[...]
    vf = vid[...].astype(jnp.float32)
    mask[...] = jnp.clip(vf, 0.0, 1.0)


def _tc_fused(xt, xv, cmf, vid, w1t, b1t, w1v, b1v, wcat, gath):
    col = lambda i: (i, 0)
    specs = [
        pl.BlockSpec((BLK, 1), col),
        pl.BlockSpec((BLK, 1), col),
        pl.BlockSpec((BLK, 1), col),
        pl.BlockSpec((BLK, 1), col),
        pl.BlockSpec((1, HID), lambda i: (0, 0)),
        pl.BlockSpec((1, HID), lambda i: (0, 0)),
        pl.BlockSpec((1, HID), lambda i: (0, 0)),
        pl.BlockSpec((1, HID), lambda i: (0, 0)),
        pl.BlockSpec((2 * HID, EMB_DIM), lambda i: (0, 0)),
        pl.BlockSpec((BLK, EMB_DIM), col),
    ]
    return pl.pallas_call(
        _tc_body,
        grid=(GRID,),
        in_specs=specs,
        out_specs=[
            pl.BlockSpec((BLK, EMB_DIM), col),
            pl.BlockSpec((BLK, 1), col),
        ],
        out_shape=[
            jax.ShapeDtypeStruct((NTOK, EMB_DIM), jnp.float32),
            jax.ShapeDtypeStruct((NTOK, 1), jnp.float32),
        ],
    )(xt, xv, cmf, vid, w1t, b1t, w1v, b1v, wcat, gath)


def kernel(time, value, var_id, category_mask, W1_t, b1_t, W2_t, W1_v, b1_v, W2_v, emb_table):
    idx_flat = var_id.reshape(NTOK)
    gath = _sc_gather(emb_table, idx_flat)

    xt = time.reshape(NTOK, 1)
    xv = value.reshape(NTOK, 1)
    cmf = category_mask.astype(jnp.float32).reshape(NTOK, 1)
    vid = var_id.reshape(NTOK, 1)
    wcat = jnp.concatenate([W2_t, W2_v], axis=0)
    sum_flat, mask_flat = _tc_fused(
        xt, xv, cmf, vid,
        W1_t, b1_t.reshape(1, HID), W1_v, b1_v.reshape(1, HID),
        wcat, gath,
    )
    return sum_flat.reshape(B, L, EMB_DIM), mask_flat.reshape(B, L)


# speedup vs baseline: 2.5928x; 2.0534x over previous
import functools

import jax
import jax.numpy as jnp
from jax import lax
from jax.experimental import pallas as pl
from jax.experimental.pallas import tpu as pltpu
from jax.experimental.pallas import tpu_sc as plsc

B, L = 4096, 200
NTOK = B * L
EMB_DIM = 64
HID = 8

NC, NS = 2, 16
NW = NC * NS
TOK_PER_W = NTOK // NW
CHUNK = 1024
STREAM = 128
N_STEPS = TOK_PER_W // CHUNK
N_SUB = CHUNK // STREAM


def _sc_gather_body(table_hbm, idx_hbm, out_hbm, idx_v, rows_v, sem):
    wid = lax.axis_index("s") * NC + lax.axis_index("c")
    base = wid * TOK_PER_W

    def step(i, _):
        off = base + i * CHUNK
        pltpu.sync_copy(idx_hbm.at[pl.ds(off, CHUNK)], idx_v)
        copies = []
        for j in range(N_SUB):
            copies.append(
                pltpu.async_copy(
                    table_hbm.at[idx_v.at[pl.ds(j * STREAM, STREAM)]],
                    rows_v.at[pl.ds(j * STREAM, STREAM)],
                    sem,
                )
            )
        for c in copies:
            c.wait()
        pltpu.sync_copy(rows_v, out_hbm.at[pl.ds(off, CHUNK)])
        return ()

    lax.fori_loop(0, N_STEPS, step, (), unroll=False)


def _sc_gather(table, idx_flat):
    mesh = plsc.VectorSubcoreMesh(core_axis_name="c", subcore_axis_name="s")
    k = functools.partial(
        pl.kernel,
        mesh=mesh,
        out_type=jax.ShapeDtypeStruct((NTOK, EMB_DIM), jnp.float32),
        scratch_types=[
            pltpu.VMEM((CHUNK,), jnp.int32),
            pltpu.VMEM((CHUNK, EMB_DIM), jnp.float32),
            pltpu.SemaphoreType.DMA,
        ],
        compiler_params=pltpu.CompilerParams(use_tc_tiling_on_sc=False),
    )(_sc_gather_body)
    return k(table, idx_flat)


BLK = 2048
GRID = NTOK // BLK


def _tc_body(xt, xv, cmf, vid, w1t, b1t, w1v, b1v, wcat, gath, out, mask):
    t = xt[0]
    v = xv[0]
    h_t = jnp.tanh(t * w1t[...] + b1t[...])
    h_v = jnp.tanh(v * w1v[...] + b1v[...]) * cmf[0]
    h = jnp.concatenate([h_t, h_v], axis=0)
    cve = lax.dot_general(
        h, wcat[...], (((0,), (0,)), ((), ())),
        preferred_element_type=jnp.float32,
    )
    out[...] = cve + gath[...]
    vf = vid[0].astype(jnp.float32)
    mask[0] = jnp.clip(vf, 0.0, 1.0)


def _tc_fused(xt, xv, cmf, vid, w1t, b1t, w1v, b1v, wcat, gath):
    row = lambda i: (i, 0)
    zero = lambda i: (0, 0)
    row3 = lambda i: (i, 0, 0)
    specs = [
        pl.BlockSpec((1, 1, BLK), row3),
        pl.BlockSpec((1, 1, BLK), row3),
        pl.BlockSpec((1, 1, BLK), row3),
        pl.BlockSpec((1, 1, BLK), row3),
        pl.BlockSpec((HID, 1), zero),
        pl.BlockSpec((HID, 1), zero),
        pl.BlockSpec((HID, 1), zero),
        pl.BlockSpec((HID, 1), zero),
        pl.BlockSpec((2 * HID, EMB_DIM), zero),
        pl.BlockSpec((BLK, EMB_DIM), row),
    ]
    return pl.pallas_call(
        _tc_body,
        grid=(GRID,),
        in_specs=specs,
        out_specs=[
            pl.BlockSpec((BLK, EMB_DIM), row),
            pl.BlockSpec((1, 1, BLK), row3),
        ],
        out_shape=[
            jax.ShapeDtypeStruct((NTOK, EMB_DIM), jnp.float32),
            jax.ShapeDtypeStruct((GRID, 1, BLK), jnp.float32),
        ],
    )(xt, xv, cmf, vid, w1t, b1t, w1v, b1v, wcat, gath)


def kernel(time, value, var_id, category_mask, W1_t, b1_t, W2_t, W1_v, b1_v, W2_v, emb_table):
    idx_flat = var_id.reshape(NTOK)
    gath = _sc_gather(emb_table, idx_flat)

    xt = time.reshape(GRID, 1, BLK)
    xv = value.reshape(GRID, 1, BLK)
    cmf = category_mask.astype(jnp.float32).reshape(GRID, 1, BLK)
    vid = var_id.reshape(GRID, 1, BLK)
    wcat = jnp.concatenate([W2_t, W2_v], axis=0)
    sum_flat, mask_flat = _tc_fused(
        xt, xv, cmf, vid,
        W1_t.reshape(HID, 1), b1_t.reshape(HID, 1),
        W1_v.reshape(HID, 1), b1_v.reshape(HID, 1),
        wcat, gath,
    )
    return sum_flat.reshape(B, L, EMB_DIM), mask_flat.reshape(B, L)
